# Initial kernel scaffold; baseline (speedup 1.0000x reference)
import jax
import jax.numpy as jnp
from jax.experimental import pallas as pl


def kernel(x0, noise, y, A, C, u):
    raise NotImplementedError("write your pallas kernel here")


# speedup vs baseline: 3.5427x; 3.5427x over previous
import functools

import jax
import jax.numpy as jnp
from jax import lax
from jax.experimental import pallas as pl

T_STEPS = 8
ESS_THRESHOLD = 2048.0


def _mm_body(x_ref, a_ref, c_ref, z_ref, pred_ref):
    x = x_ref[0]
    a = a_ref[...]
    c = c_ref[...]
    z_ref[0] = lax.dot_general(x, a, (((1,), (1,)), ((), ())))
    pred_ref[0] = lax.dot_general(x, c, (((1,), (1,)), ((), ())))


@jax.jit
def _mm(x, A, C):
    B, N, d = x.shape
    d_obs = C.shape[0]
    return pl.pallas_call(
        _mm_body,
        grid=(B,),
        in_specs=[
            pl.BlockSpec((1, N, d), lambda b: (b, 0, 0)),
            pl.BlockSpec((d, d), lambda b: (0, 0)),
            pl.BlockSpec((d_obs, d), lambda b: (0, 0)),
        ],
        out_specs=[
            pl.BlockSpec((1, N, d), lambda b: (b, 0, 0)),
            pl.BlockSpec((1, N, d_obs), lambda b: (b, 0, 0)),
        ],
        out_shape=[
            jax.ShapeDtypeStruct((B, N, d), jnp.float32),
            jax.ShapeDtypeStruct((B, N, d_obs), jnp.float32),
        ],
    )(x, A, C)


def _norm_log(lw):
    return lw - jax.scipy.special.logsumexp(lw, axis=-1, keepdims=True)


def _resample_idx(lnw, u_t):
    w = jnp.exp(lnw)
    cdf = jnp.cumsum(w, axis=-1)
    N = lnw.shape[-1]
    pos = (u_t[:, None] + jnp.arange(N, dtype=jnp.float32)[None, :]) / N
    idx = jax.vmap(lambda c, p: jnp.searchsorted(c, p))(cdf, pos)
    return jnp.clip(idx, 0, N - 1)


def kernel(x0, noise, y, A, C, u):
    B, N, d = x0.shape
    Tn = noise.shape[0]

    x_t = x0
    _, pred0 = _mm(x_t, A, C)
    lw = -0.5 * jnp.sum((pred0 - y[0][:, None, :]) ** 2, axis=-1)
    lnw = _norm_log(lw)
    means = [jnp.sum(jnp.exp(lnw)[..., None] * x_t, axis=1)]

    for t in range(1, Tn + 1):
        ess = 1.0 / jnp.sum(jnp.exp(2.0 * lnw), axis=-1)
        idx = _resample_idx(lnw, u[t - 1])
        x_res = jnp.take_along_axis(x_t, idx[..., None], axis=1)
        lw_res = jnp.full_like(lnw, -jnp.log(float(N)))
        mask = ess < ESS_THRESHOLD
        x_prev = jnp.where(mask[:, None, None], x_res, x_t)
        lw_prev = jnp.where(mask[:, None], lw_res, lnw)

        z, _ = _mm(x_prev, A, C)
        x_t = z + noise[t - 1]
        _, pred = _mm(x_t, A, C)
        lw = lw_prev + (-0.5 * jnp.sum((pred - y[t][:, None, :]) ** 2, axis=-1))
        lnw = _norm_log(lw)
        means.append(jnp.sum(jnp.exp(lnw)[..., None] * x_t, axis=1))

    return jnp.stack(means, axis=0)


# speedup vs baseline: 9.9650x; 2.8128x over previous
import functools

import jax
import jax.numpy as jnp
from jax import lax
from jax.experimental import pallas as pl
from jax.experimental.pallas import tpu as pltpu
from jax.experimental.pallas import tpu_sc as plsc

_ESS_THRESHOLD = 2048.0


def _init_body(x_ref, c_ref, p_ref):
    p_ref[0] = lax.dot_general(x_ref[0], c_ref[...], (((1,), (1,)), ((), ())))


@jax.jit
def _tc_init(x0, C):
    B, N, d = x0.shape
    d_obs = C.shape[0]
    return pl.pallas_call(
        _init_body,
        grid=(B,),
        in_specs=[
            pl.BlockSpec((1, N, d), lambda b: (b, 0, 0)),
            pl.BlockSpec((d_obs, d), lambda b: (0, 0)),
        ],
        out_specs=pl.BlockSpec((1, N, d_obs), lambda b: (b, 0, 0)),
        out_shape=jax.ShapeDtypeStruct((B, N, d_obs), jnp.float32),
    )(x0, C)


def _step_body(xp_ref, n_ref, a_ref, c_ref, xt_ref, p_ref):
    xt = lax.dot_general(xp_ref[0], a_ref[...], (((1,), (1,)), ((), ()))) + n_ref[0]
    xt_ref[0] = xt
    p_ref[0] = lax.dot_general(xt, c_ref[...], (((1,), (1,)), ((), ())))


@jax.jit
def _tc_step(x_prev, noise_t, A, C):
    B, N, d = x_prev.shape
    d_obs = C.shape[0]
    return pl.pallas_call(
        _step_body,
        grid=(B,),
        in_specs=[
            pl.BlockSpec((1, N, d), lambda b: (b, 0, 0)),
            pl.BlockSpec((1, N, d), lambda b: (b, 0, 0)),
            pl.BlockSpec((d, d), lambda b: (0, 0)),
            pl.BlockSpec((d_obs, d), lambda b: (0, 0)),
        ],
        out_specs=[
            pl.BlockSpec((1, N, d), lambda b: (b, 0, 0)),
            pl.BlockSpec((1, N, d_obs), lambda b: (b, 0, 0)),
        ],
        out_shape=[
            jax.ShapeDtypeStruct((B, N, d), jnp.float32),
            jax.ShapeDtypeStruct((B, N, d_obs), jnp.float32),
        ],
    )(x_prev, noise_t, A, C)


def _mean_body(w_ref, x_ref, m_ref):
    m_ref[0, 0] = jnp.sum(w_ref[0, 0][:, None] * x_ref[0], axis=0)


@jax.jit
def _tc_mean(w, x):
    B, N, d = x.shape
    return pl.pallas_call(
        _mean_body,
        grid=(B,),
        in_specs=[
            pl.BlockSpec((1, 1, N), lambda b: (b, 0, 0)),
            pl.BlockSpec((1, N, d), lambda b: (b, 0, 0)),
        ],
        out_specs=pl.BlockSpec((1, 1, d), lambda b: (b, 0, 0)),
        out_shape=jax.ShapeDtypeStruct((B, 1, d), jnp.float32),
    )(w.reshape(B, 1, N), x)[:, 0, :]


_SC_N = 4096
_SC_D = 32


def _sc_body(cdf_hbm, u_hbm, mask_hbm, xt_hbm, out_hbm,
             cdf_v, idx_v, u_v, m_v, buf0, buf1, sem0, sem1):
    wid = lax.axis_index("s") * 2 + lax.axis_index("c")
    b = wid // 2
    h = wid % 2
    half_n = _SC_N // 2

    pltpu.sync_copy(cdf_hbm.at[b], cdf_v)
    pltpu.sync_copy(u_hbm.at[b], u_v)
    pltpu.sync_copy(mask_hbm.at[b], m_v)
    uvec = u_v[...]
    mvec = m_v[...]
    iota16 = lax.iota(jnp.int32, 16)
    inv_n = jnp.full((16,), 1.0 / _SC_N, jnp.float32)
    zero16 = jnp.zeros((16,), jnp.int32)
    ones16 = jnp.ones((16,), jnp.int32)
    n16 = jnp.full((16,), _SC_N, jnp.int32)
    nm1_16 = jnp.full((16,), _SC_N - 1, jnp.int32)
    half16 = jnp.full((16,), 0.5, jnp.float32)
    qbase = jnp.full((16,), h * half_n, jnp.int32) + iota16

    def bs_body(jj, carry):
        gq = qbase + jnp.full((16,), jj * 16, jnp.int32)
        posq = (uvec + gq.astype(jnp.float32)) * inv_n
        lo = zero16
        hi = n16
        for _ in range(13):
            mid = lax.shift_right_arithmetic(lo + hi, ones16)
            cm = plsc.load_gather(cdf_v, [jnp.minimum(mid, nm1_16)])
            cond = cm < posq
            lo = jnp.where(cond, mid + ones16, lo)
            hi = jnp.where(cond, hi, mid)
        idxq = jnp.minimum(lo, nm1_16)
        idxf = jnp.where(mvec > half16, idxq, gq)
        idx_v[pl.ds(jj * 16, 16)] = idxf
        return carry
    lax.fori_loop(0, half_n // 16, bs_body, jnp.int32(0))

    src = xt_hbm.at[b]
    base = h * half_n
    bufs = (buf0, buf1)
    sems = (sem0, sem1)
    n_chunks = half_n // 128
    cps = []
    for j2 in range(n_chunks):
        bse = base + j2 * 128
        cp = pltpu.async_copy(src.at[idx_v.at[pl.ds(j2 * 128, 128)]], bufs[j2 % 2], sems[j2 % 2])
        cps.append(cp)
        if j2 >= 1:
            cps[j2 - 1].wait()
            pbse = base + (j2 - 1) * 128
            pltpu.sync_copy(bufs[(j2 - 1) % 2], out_hbm.at[b, pl.ds(pbse, 128)])
    cps[-1].wait()
    pltpu.sync_copy(bufs[(n_chunks - 1) % 2],
                    out_hbm.at[b, pl.ds(base + (n_chunks - 1) * 128, 128)])


@jax.jit
def _sc_resample(cdf, uu, maskf, xt):
    B, N, d = xt.shape
    kern = functools.partial(
        pl.kernel,
        out_type=jax.ShapeDtypeStruct((B, N, d), jnp.float32),
        mesh=plsc.VectorSubcoreMesh(core_axis_name="c", subcore_axis_name="s"),
        compiler_params=pltpu.CompilerParams(needs_layout_passes=False,
                                             use_tc_tiling_on_sc=False),
        scratch_types=[
            pltpu.VMEM((_SC_N,), jnp.float32),
            pltpu.VMEM((_SC_N // 2,), jnp.int32),
            pltpu.VMEM((16,), jnp.float32),
            pltpu.VMEM((16,), jnp.float32),
            pltpu.VMEM((128, _SC_D), jnp.float32),
            pltpu.VMEM((128, _SC_D), jnp.float32),
            pltpu.SemaphoreType.DMA,
            pltpu.SemaphoreType.DMA,
        ],
    )(_sc_body)
    return kern(cdf, uu, maskf, xt)


def _norm_log(lw):
    return lw - jax.scipy.special.logsumexp(lw, axis=-1, keepdims=True)


def kernel(x0, noise, y, A, C, u):
    B, N, d = x0.shape
    Tn = noise.shape[0]

    x_t = x0
    pred = _tc_init(x0, C)
    lw = -0.5 * jnp.sum((pred - y[0][:, None, :]) ** 2, axis=-1)
    lnw = _norm_log(lw)
    w = jnp.exp(lnw)
    means = [_tc_mean(w, x_t)]

    for t in range(1, Tn + 1):
        ess = 1.0 / jnp.sum(jnp.exp(2.0 * lnw), axis=-1)
        mask = ess < _ESS_THRESHOLD
        cdf = jnp.cumsum(w, axis=-1)
        lw_res = jnp.full_like(lnw, -jnp.log(float(N)))
        lw_prev = jnp.where(mask[:, None], lw_res, lnw)

        uu = jnp.broadcast_to(u[t - 1][:, None], (B, 16))
        maskf = jnp.broadcast_to(mask[:, None].astype(jnp.float32), (B, 16))
        x_prev = _sc_resample(cdf, uu, maskf, x_t)

        x_t, pred = _tc_step(x_prev, noise[t - 1], A, C)
        lw = lw_prev + (-0.5 * jnp.sum((pred - y[t][:, None, :]) ** 2, axis=-1))
        lnw = _norm_log(lw)
        w = jnp.exp(lnw)
        means.append(_tc_mean(w, x_t))

    return jnp.stack(means, axis=0)


# speedup vs baseline: 10.6978x; 1.0735x over previous
import functools

import jax
import jax.numpy as jnp
from jax import lax
from jax.experimental import pallas as pl
from jax.experimental.pallas import tpu as pltpu
from jax.experimental.pallas import tpu_sc as plsc

_ESS_THRESHOLD = 2048.0


def _init_body(x_ref, c_ref, p_ref):
    p_ref[0] = lax.dot_general(x_ref[0], c_ref[...], (((1,), (1,)), ((), ())))


@jax.jit
def _tc_init(x0, C):
    B, N, d = x0.shape
    d_obs = C.shape[0]
    return pl.pallas_call(
        _init_body,
        grid=(B,),
        in_specs=[
            pl.BlockSpec((1, N, d), lambda b: (b, 0, 0)),
            pl.BlockSpec((d_obs, d), lambda b: (0, 0)),
        ],
        out_specs=pl.BlockSpec((1, N, d_obs), lambda b: (b, 0, 0)),
        out_shape=jax.ShapeDtypeStruct((B, N, d_obs), jnp.float32),
    )(x0, C)


def _step_body(xp_ref, n_ref, a_ref, c_ref, wo_ref, xo_ref, xt_ref, p_ref, m_ref):
    xt = lax.dot_general(xp_ref[0], a_ref[...], (((1,), (1,)), ((), ()))) + n_ref[0]
    xt_ref[0] = xt
    p_ref[0] = lax.dot_general(xt, c_ref[...], (((1,), (1,)), ((), ())))
    m_ref[0, 0] = jnp.sum(wo_ref[0, 0][:, None] * xo_ref[0], axis=0)


@jax.jit
def _tc_step(x_prev, noise_t, A, C, w_old, x_old):
    B, N, d = x_prev.shape
    d_obs = C.shape[0]
    return pl.pallas_call(
        _step_body,
        grid=(B,),
        in_specs=[
            pl.BlockSpec((1, N, d), lambda b: (b, 0, 0)),
            pl.BlockSpec((1, N, d), lambda b: (b, 0, 0)),
            pl.BlockSpec((d, d), lambda b: (0, 0)),
            pl.BlockSpec((d_obs, d), lambda b: (0, 0)),
            pl.BlockSpec((1, 1, N), lambda b: (b, 0, 0)),
            pl.BlockSpec((1, N, d), lambda b: (b, 0, 0)),
        ],
        out_specs=[
            pl.BlockSpec((1, N, d), lambda b: (b, 0, 0)),
            pl.BlockSpec((1, N, d_obs), lambda b: (b, 0, 0)),
            pl.BlockSpec((1, 1, d), lambda b: (b, 0, 0)),
        ],
        out_shape=[
            jax.ShapeDtypeStruct((B, N, d), jnp.float32),
            jax.ShapeDtypeStruct((B, N, d_obs), jnp.float32),
            jax.ShapeDtypeStruct((B, 1, d), jnp.float32),
        ],
    )(x_prev, noise_t, A, C, w_old.reshape(B, 1, N), x_old)


def _mean_body(w_ref, x_ref, m_ref):
    m_ref[0, 0] = jnp.sum(w_ref[0, 0][:, None] * x_ref[0], axis=0)


@jax.jit
def _tc_mean(w, x):
    B, N, d = x.shape
    return pl.pallas_call(
        _mean_body,
        grid=(B,),
        in_specs=[
            pl.BlockSpec((1, 1, N), lambda b: (b, 0, 0)),
            pl.BlockSpec((1, N, d), lambda b: (b, 0, 0)),
        ],
        out_specs=pl.BlockSpec((1, 1, d), lambda b: (b, 0, 0)),
        out_shape=jax.ShapeDtypeStruct((B, 1, d), jnp.float32),
    )(w.reshape(B, 1, N), x)[:, 0, :]


_SC_N = 4096
_SC_D = 32


def _sc_body(cdf_hbm, u_hbm, mask_hbm, xt_hbm, out_hbm,
             cdf_v, idx_v, u_v, m_v, buf0, buf1, sem0, sem1):
    wid = lax.axis_index("s") * 2 + lax.axis_index("c")
    b = wid // 2
    h = wid % 2
    half_n = _SC_N // 2

    pltpu.sync_copy(cdf_hbm.at[b], cdf_v)
    pltpu.sync_copy(u_hbm.at[b], u_v)
    pltpu.sync_copy(mask_hbm.at[b], m_v)
    uvec = u_v[...]
    mvec = m_v[...]
    iota16 = lax.iota(jnp.int32, 16)
    inv_n = jnp.full((16,), 1.0 / _SC_N, jnp.float32)
    zero16 = jnp.zeros((16,), jnp.int32)
    ones16 = jnp.ones((16,), jnp.int32)
    n16 = jnp.full((16,), _SC_N, jnp.int32)
    nm1_16 = jnp.full((16,), _SC_N - 1, jnp.int32)
    half16 = jnp.full((16,), 0.5, jnp.float32)
    qbase = jnp.full((16,), h * half_n, jnp.int32) + iota16

    @plsc.parallel_loop(0, half_n // 16, unroll=4)
    def bs_body(jj):
        gq = qbase + jnp.full((16,), jj * 16, jnp.int32)
        posq = (uvec + gq.astype(jnp.float32)) * inv_n
        lo = zero16
        hi = n16
        for _ in range(13):
            mid = lax.shift_right_arithmetic(lo + hi, ones16)
            cm = plsc.load_gather(cdf_v, [jnp.minimum(mid, nm1_16)])
            cond = cm < posq
            lo = jnp.where(cond, mid + ones16, lo)
            hi = jnp.where(cond, hi, mid)
        idxq = jnp.minimum(lo, nm1_16)
        idxf = jnp.where(mvec > half16, idxq, gq)
        idx_v[pl.ds(jj * 16, 16)] = idxf

    src = xt_hbm.at[b]
    base = h * half_n
    bufs = (buf0, buf1)
    sems = (sem0, sem1)
    n_chunks = half_n // 128
    cps = []
    for j2 in range(n_chunks):
        bse = base + j2 * 128
        cp = pltpu.async_copy(src.at[idx_v.at[pl.ds(j2 * 128, 128)]], bufs[j2 % 2], sems[j2 % 2])
        cps.append(cp)
        if j2 >= 1:
            cps[j2 - 1].wait()
            pbse = base + (j2 - 1) * 128
            pltpu.sync_copy(bufs[(j2 - 1) % 2], out_hbm.at[b, pl.ds(pbse, 128)])
    cps[-1].wait()
    pltpu.sync_copy(bufs[(n_chunks - 1) % 2],
                    out_hbm.at[b, pl.ds(base + (n_chunks - 1) * 128, 128)])


@jax.jit
def _sc_resample(cdf, uu, maskf, xt):
    B, N, d = xt.shape
    kern = functools.partial(
        pl.kernel,
        out_type=jax.ShapeDtypeStruct((B, N, d), jnp.float32),
        mesh=plsc.VectorSubcoreMesh(core_axis_name="c", subcore_axis_name="s"),
        compiler_params=pltpu.CompilerParams(needs_layout_passes=False,
                                             use_tc_tiling_on_sc=False),
        scratch_types=[
            pltpu.VMEM((_SC_N,), jnp.float32),
            pltpu.VMEM((_SC_N // 2,), jnp.int32),
            pltpu.VMEM((16,), jnp.float32),
            pltpu.VMEM((16,), jnp.float32),
            pltpu.VMEM((128, _SC_D), jnp.float32),
            pltpu.VMEM((128, _SC_D), jnp.float32),
            pltpu.SemaphoreType.DMA,
            pltpu.SemaphoreType.DMA,
        ],
    )(_sc_body)
    return kern(cdf, uu, maskf, xt)


def _norm_log(lw):
    return lw - jax.scipy.special.logsumexp(lw, axis=-1, keepdims=True)


def kernel(x0, noise, y, A, C, u):
    B, N, d = x0.shape
    Tn = noise.shape[0]

    x_t = x0
    pred = _tc_init(x0, C)
    lw = -0.5 * jnp.sum((pred - y[0][:, None, :]) ** 2, axis=-1)
    lnw = _norm_log(lw)
    w = jnp.exp(lnw)
    means = []

    for t in range(1, Tn + 1):
        ess = 1.0 / jnp.sum(jnp.exp(2.0 * lnw), axis=-1)
        mask = ess < _ESS_THRESHOLD
        cdf = jnp.cumsum(w, axis=-1)
        lw_res = jnp.full_like(lnw, -jnp.log(float(N)))
        lw_prev = jnp.where(mask[:, None], lw_res, lnw)

        uu = jnp.broadcast_to(u[t - 1][:, None], (B, 16))
        maskf = jnp.broadcast_to(mask[:, None].astype(jnp.float32), (B, 16))
        x_prev = _sc_resample(cdf, uu, maskf, x_t)

        x_new, pred, m_old = _tc_step(x_prev, noise[t - 1], A, C, w, x_t)
        means.append(m_old[:, 0, :])
        x_t = x_new
        lw = lw_prev + (-0.5 * jnp.sum((pred - y[t][:, None, :]) ** 2, axis=-1))
        lnw = _norm_log(lw)
        w = jnp.exp(lnw)

    means.append(_tc_mean(w, x_t))
    return jnp.stack(means, axis=0)


# speedup vs baseline: 10.9529x; 1.0239x over previous
import functools

import jax
import jax.numpy as jnp
from jax import lax
from jax.experimental import pallas as pl
from jax.experimental.pallas import tpu as pltpu
from jax.experimental.pallas import tpu_sc as plsc

_ESS_THRESHOLD = 2048.0


def _init_body(x_ref, c_ref, p_ref):
    p_ref[0] = lax.dot_general(x_ref[0], c_ref[...], (((1,), (1,)), ((), ())))


@jax.jit
def _tc_init(x0, C):
    B, N, d = x0.shape
    d_obs = C.shape[0]
    return pl.pallas_call(
        _init_body,
        grid=(B,),
        in_specs=[
            pl.BlockSpec((1, N, d), lambda b: (b, 0, 0)),
            pl.BlockSpec((d_obs, d), lambda b: (0, 0)),
        ],
        out_specs=pl.BlockSpec((1, N, d_obs), lambda b: (b, 0, 0)),
        out_shape=jax.ShapeDtypeStruct((B, N, d_obs), jnp.float32),
    )(x0, C)


def _step_body(xp_ref, n_ref, a_ref, c_ref, wo_ref, xo_ref, xt_ref, p_ref, m_ref):
    xt = lax.dot_general(xp_ref[0], a_ref[...], (((1,), (1,)), ((), ()))) + n_ref[0]
    xt_ref[0] = xt
    p_ref[0] = lax.dot_general(xt, c_ref[...], (((1,), (1,)), ((), ())))
    m_ref[0, 0] = jnp.sum(wo_ref[0, 0][:, None] * xo_ref[0], axis=0)


@jax.jit
def _tc_step(x_prev, noise_t, A, C, w_old, x_old):
    B, N, d = x_prev.shape
    d_obs = C.shape[0]
    return pl.pallas_call(
        _step_body,
        grid=(B,),
        in_specs=[
            pl.BlockSpec((1, N, d), lambda b: (b, 0, 0)),
            pl.BlockSpec((1, N, d), lambda b: (b, 0, 0)),
            pl.BlockSpec((d, d), lambda b: (0, 0)),
            pl.BlockSpec((d_obs, d), lambda b: (0, 0)),
            pl.BlockSpec((1, 1, N), lambda b: (b, 0, 0)),
            pl.BlockSpec((1, N, d), lambda b: (b, 0, 0)),
        ],
        out_specs=[
            pl.BlockSpec((1, N, d), lambda b: (b, 0, 0)),
            pl.BlockSpec((1, N, d_obs), lambda b: (b, 0, 0)),
            pl.BlockSpec((1, 1, d), lambda b: (b, 0, 0)),
        ],
        out_shape=[
            jax.ShapeDtypeStruct((B, N, d), jnp.float32),
            jax.ShapeDtypeStruct((B, N, d_obs), jnp.float32),
            jax.ShapeDtypeStruct((B, 1, d), jnp.float32),
        ],
    )(x_prev, noise_t, A, C, w_old.reshape(B, 1, N), x_old)


def _mean_body(w_ref, x_ref, m_ref):
    m_ref[0, 0] = jnp.sum(w_ref[0, 0][:, None] * x_ref[0], axis=0)


@jax.jit
def _tc_mean(w, x):
    B, N, d = x.shape
    return pl.pallas_call(
        _mean_body,
        grid=(B,),
        in_specs=[
            pl.BlockSpec((1, 1, N), lambda b: (b, 0, 0)),
            pl.BlockSpec((1, N, d), lambda b: (b, 0, 0)),
        ],
        out_specs=pl.BlockSpec((1, 1, d), lambda b: (b, 0, 0)),
        out_shape=jax.ShapeDtypeStruct((B, 1, d), jnp.float32),
    )(w.reshape(B, 1, N), x)[:, 0, :]


_SC_N = 4096
_SC_D = 32


def _sc_body(cdf_hbm, u_hbm, mask_hbm, xt_hbm, out_hbm,
             cdf_v, idx_v, u_v, m_v, *bufs_sems):
    bufs = bufs_sems[:16]
    sem_g, sem_s = bufs_sems[16], bufs_sems[17]
    wid = lax.axis_index("s") * 2 + lax.axis_index("c")
    b = wid // 2
    h = wid % 2
    half_n = _SC_N // 2

    pltpu.sync_copy(cdf_hbm.at[b], cdf_v)
    pltpu.sync_copy(u_hbm.at[b], u_v)
    pltpu.sync_copy(mask_hbm.at[b], m_v)
    uvec = u_v[...]
    mvec = m_v[...]
    iota16 = lax.iota(jnp.int32, 16)
    inv_n = jnp.full((16,), 1.0 / _SC_N, jnp.float32)
    zero16 = jnp.zeros((16,), jnp.int32)
    ones16 = jnp.ones((16,), jnp.int32)
    n16 = jnp.full((16,), _SC_N, jnp.int32)
    nm1_16 = jnp.full((16,), _SC_N - 1, jnp.int32)
    half16 = jnp.full((16,), 0.5, jnp.float32)
    qbase = jnp.full((16,), h * half_n, jnp.int32) + iota16

    @plsc.parallel_loop(0, half_n // 16, unroll=8)
    def bs_body(jj):
        gq = qbase + jnp.full((16,), jj * 16, jnp.int32)
        posq = (uvec + gq.astype(jnp.float32)) * inv_n
        lo = zero16
        hi = n16
        for _ in range(13):
            mid = lax.shift_right_arithmetic(lo + hi, ones16)
            cm = plsc.load_gather(cdf_v, [jnp.minimum(mid, nm1_16)])
            cond = cm < posq
            lo = jnp.where(cond, mid + ones16, lo)
            hi = jnp.where(cond, hi, mid)
        idxq = jnp.minimum(lo, nm1_16)
        idxf = jnp.where(mvec > half16, idxq, gq)
        idx_v[pl.ds(jj * 16, 16)] = idxf

    src = xt_hbm.at[b]
    base = h * half_n
    n_chunks = half_n // 128
    cps = []
    for j2 in range(n_chunks):
        cps.append(pltpu.async_copy(src.at[idx_v.at[pl.ds(j2 * 128, 128)]],
                                    bufs[j2], sem_g))
    sts = []
    for j2 in range(n_chunks):
        cps[j2].wait()
        sts.append(pltpu.async_copy(bufs[j2], out_hbm.at[b, pl.ds(base + j2 * 128, 128)],
                                    sem_s))
    for st in sts:
        st.wait()


@jax.jit
def _sc_resample(cdf, uu, maskf, xt):
    B, N, d = xt.shape
    kern = functools.partial(
        pl.kernel,
        out_type=jax.ShapeDtypeStruct((B, N, d), jnp.float32),
        mesh=plsc.VectorSubcoreMesh(core_axis_name="c", subcore_axis_name="s"),
        compiler_params=pltpu.CompilerParams(needs_layout_passes=False,
                                             use_tc_tiling_on_sc=False),
        scratch_types=[
            pltpu.VMEM((_SC_N,), jnp.float32),
            pltpu.VMEM((_SC_N // 2,), jnp.int32),
            pltpu.VMEM((16,), jnp.float32),
            pltpu.VMEM((16,), jnp.float32),
        ] + [pltpu.VMEM((128, _SC_D), jnp.float32) for _ in range(16)] + [
            pltpu.SemaphoreType.DMA,
            pltpu.SemaphoreType.DMA,
        ],
    )(_sc_body)
    return kern(cdf, uu, maskf, xt)


def _norm_log(lw):
    return lw - jax.scipy.special.logsumexp(lw, axis=-1, keepdims=True)


def kernel(x0, noise, y, A, C, u):
    B, N, d = x0.shape
    Tn = noise.shape[0]

    x_t = x0
    pred = _tc_init(x0, C)
    lw = -0.5 * jnp.sum((pred - y[0][:, None, :]) ** 2, axis=-1)
    lnw = _norm_log(lw)
    w = jnp.exp(lnw)
    means = []

    for t in range(1, Tn + 1):
        ess = 1.0 / jnp.sum(jnp.exp(2.0 * lnw), axis=-1)
        mask = ess < _ESS_THRESHOLD
        cdf = jnp.cumsum(w, axis=-1)
        lw_res = jnp.full_like(lnw, -jnp.log(float(N)))
        lw_prev = jnp.where(mask[:, None], lw_res, lnw)

        uu = jnp.broadcast_to(u[t - 1][:, None], (B, 16))
        maskf = jnp.broadcast_to(mask[:, None].astype(jnp.float32), (B, 16))
        x_prev = _sc_resample(cdf, uu, maskf, x_t)

        x_new, pred, m_old = _tc_step(x_prev, noise[t - 1], A, C, w, x_t)
        means.append(m_old[:, 0, :])
        x_t = x_new
        lw = lw_prev + (-0.5 * jnp.sum((pred - y[t][:, None, :]) ** 2, axis=-1))
        lnw = _norm_log(lw)
        w = jnp.exp(lnw)

    means.append(_tc_mean(w, x_t))
    return jnp.stack(means, axis=0)
